# Initial kernel scaffold; baseline (speedup 1.0000x reference)
import jax
import jax.numpy as jnp
from jax.experimental import pallas as pl


def kernel(pos_xyz, cel_mat, sft_cel, idx_i, idx_j, idx_s):
    raise NotImplementedError("write your pallas kernel here")


# speedup vs baseline: 18.6723x; 18.6723x over previous
import functools

import jax
import jax.numpy as jnp
from jax import lax
from jax.experimental import pallas as pl
from jax.experimental.pallas import tpu as pltpu
from jax.experimental.pallas import tpu_sc as plsc

_NC = 2
_NS = 16
_SUB = 80
_NSUB = 25
_CHUNK = _SUB * _NSUB
_RC2 = 36.0


def _make_sc_kernel(E, N):
    NW = _NC * _NS
    per_worker = E // NW
    n_chunks = per_worker // _CHUNK
    n_rows = E // _SUB

    mesh = plsc.VectorSubcoreMesh(
        core_axis_name="c", subcore_axis_name="s", num_cores=_NC
    )

    @functools.partial(
        pl.kernel,
        out_type=[
            jax.ShapeDtypeStruct((E, 3), jnp.float32),
            jax.ShapeDtypeStruct((E,), jnp.float32),
            jax.ShapeDtypeStruct((E,), jnp.int32),
        ],
        mesh=mesh,
        compiler_params=pltpu.CompilerParams(
            needs_layout_passes=False, use_tc_tiling_on_sc=False
        ),
        scratch_types=[
            pltpu.VMEM((32, 3), jnp.float32),
            pltpu.VMEM((_NSUB, _SUB), jnp.int32),
            pltpu.VMEM((_NSUB, _SUB), jnp.int32),
            pltpu.VMEM((_CHUNK,), jnp.int32),
            pltpu.VMEM((_CHUNK, 16), jnp.float32),
            pltpu.VMEM((_CHUNK, 16), jnp.float32),
            pltpu.VMEM((_CHUNK, 3), jnp.float32),
            pltpu.VMEM((_CHUNK,), jnp.float32),
            pltpu.VMEM((_CHUNK,), jnp.int32),
            pltpu.SemaphoreType.DMA,
        ],
    )
    def sc_kernel(pos4, sft_xyz, ii, jj, ss,
                  vec_out, sod_out, mask_out,
                  sft_v, idxi_v, idxj_v, ss_v,
                  ri_v, rj_v, vec_v, sod_v, mask_v, sem):
        wid = lax.axis_index("s") * _NC + lax.axis_index("c")
        iota = lax.iota(jnp.int32, 16)
        c0 = jnp.zeros((16,), jnp.int32)
        c1 = jnp.full((16,), 1, jnp.int32)
        c2 = jnp.full((16,), 2, jnp.int32)

        pltpu.sync_copy(sft_xyz, sft_v)

        rows0 = wid * (n_chunks * _NSUB)

        def chunk_body(c, _):
            row0 = rows0 + c * _NSUB
            base = row0 * _SUB
            pltpu.sync_copy(ii.at[pl.ds(row0, _NSUB)], idxi_v)
            pltpu.sync_copy(jj.at[pl.ds(row0, _NSUB)], idxj_v)
            pltpu.sync_copy(ss.at[pl.ds(base, _CHUNK)], ss_v)

            def gstart(j, carry):
                pltpu.make_async_copy(
                    pos4.at[idxi_v.at[j]], ri_v.at[pl.ds(j * _SUB, _SUB)], sem
                ).start()
                pltpu.make_async_copy(
                    pos4.at[idxj_v.at[j]], rj_v.at[pl.ds(j * _SUB, _SUB)], sem
                ).start()
                return carry

            lax.fori_loop(0, _NSUB, gstart, 0)

            def gwait(j, carry):
                pltpu.make_async_copy(
                    pos4.at[idxi_v.at[j]], ri_v.at[pl.ds(j * _SUB, _SUB)], sem
                ).wait()
                pltpu.make_async_copy(
                    pos4.at[idxj_v.at[j]], rj_v.at[pl.ds(j * _SUB, _SUB)], sem
                ).wait()
                return carry

            lax.fori_loop(0, _NSUB, gwait, 0)

            def grp(t, carry):
                e16 = iota + t * 16
                ssv = ss_v[pl.ds(t * 16, 16)]
                xi = plsc.load_gather(ri_v, [e16, c0])
                yi = plsc.load_gather(ri_v, [e16, c1])
                zi = plsc.load_gather(ri_v, [e16, c2])
                xj = plsc.load_gather(rj_v, [e16, c0])
                yj = plsc.load_gather(rj_v, [e16, c1])
                zj = plsc.load_gather(rj_v, [e16, c2])
                sx = plsc.load_gather(sft_v, [ssv, c0])
                sy = plsc.load_gather(sft_v, [ssv, c1])
                sz = plsc.load_gather(sft_v, [ssv, c2])
                vx = (xj + sx) - xi
                vy = (yj + sy) - yi
                vz = (zj + sz) - zi
                sod = (vx * vx + vy * vy) + vz * vz
                m = sod < _RC2
                zf = jnp.zeros((16,), jnp.float32)
                sod_v[pl.ds(t * 16, 16)] = jnp.where(m, sod, zf)
                mask_v[pl.ds(t * 16, 16)] = m.astype(jnp.int32)
                plsc.store_scatter(vec_v, [e16, c0], jnp.where(m, vx, zf))
                plsc.store_scatter(vec_v, [e16, c1], jnp.where(m, vy, zf))
                plsc.store_scatter(vec_v, [e16, c2], jnp.where(m, vz, zf))
                return carry

            lax.fori_loop(0, _CHUNK // 16, grp, 0)

            pltpu.sync_copy(vec_v, vec_out.at[pl.ds(base, _CHUNK)])
            pltpu.sync_copy(sod_v, sod_out.at[pl.ds(base, _CHUNK)])
            pltpu.sync_copy(mask_v, mask_out.at[pl.ds(base, _CHUNK)])
            return _

        lax.fori_loop(0, n_chunks, chunk_body, 0)

    return sc_kernel


def kernel(pos_xyz, cel_mat, sft_cel, idx_i, idx_j, idx_s):
    N = pos_xyz.shape[0]
    E = idx_i.shape[0]
    pos4 = jnp.pad(pos_xyz.astype(jnp.float32), ((0, 0), (0, 13)))
    sft_xyz = sft_cel.astype(jnp.float32) @ cel_mat.astype(jnp.float32)
    sft_pad = jnp.pad(sft_xyz, ((0, 32 - sft_xyz.shape[0]), (0, 0)))
    ii = idx_i.astype(jnp.int32).reshape(E // _SUB, _SUB)
    jj = idx_j.astype(jnp.int32).reshape(E // _SUB, _SUB)
    ss = idx_s.astype(jnp.int32)
    sc = _make_sc_kernel(E, N)
    vec, sod, mask_i = sc(pos4, sft_pad, ii, jj, ss)
    return vec, sod, mask_i.astype(jnp.bool_)


# speedup vs baseline: 56.2606x; 3.0131x over previous
import functools

import jax
import jax.numpy as jnp
from jax import lax
from jax.experimental import pallas as pl
from jax.experimental.pallas import tpu as pltpu
from jax.experimental.pallas import tpu_sc as plsc

_NC = 2
_NS = 16
_SUB = 80
_NSUB = 25
_CHUNK = _SUB * _NSUB
_RC2 = 36.0


def _make_sc_kernel(E, N):
    NW = _NC * _NS
    per_worker = E // NW
    n_chunks = per_worker // _CHUNK
    n_rows = E // _SUB

    mesh = plsc.VectorSubcoreMesh(
        core_axis_name="c", subcore_axis_name="s", num_cores=_NC
    )

    @functools.partial(
        pl.kernel,
        out_type=[
            jax.ShapeDtypeStruct((E,), jnp.float32),
            jax.ShapeDtypeStruct((E,), jnp.float32),
            jax.ShapeDtypeStruct((E,), jnp.float32),
            jax.ShapeDtypeStruct((E,), jnp.float32),
            jax.ShapeDtypeStruct((E,), jnp.int32),
        ],
        mesh=mesh,
        compiler_params=pltpu.CompilerParams(
            needs_layout_passes=False, use_tc_tiling_on_sc=False
        ),
        scratch_types=[
            pltpu.VMEM((32, 3), jnp.float32),
            pltpu.VMEM((_NSUB, _SUB), jnp.int32),
            pltpu.VMEM((_NSUB, _SUB), jnp.int32),
            pltpu.VMEM((_CHUNK,), jnp.int32),
            pltpu.VMEM((_CHUNK, 16), jnp.float32),
            pltpu.VMEM((_CHUNK, 16), jnp.float32),
            pltpu.VMEM((_CHUNK,), jnp.float32),
            pltpu.VMEM((_CHUNK,), jnp.float32),
            pltpu.VMEM((_CHUNK,), jnp.float32),
            pltpu.VMEM((_CHUNK,), jnp.float32),
            pltpu.VMEM((_CHUNK,), jnp.int32),
            pltpu.SemaphoreType.DMA,
        ],
    )
    def sc_kernel(pos4, sft_xyz, ii, jj, ss,
                  vx_out, vy_out, vz_out, sod_out, mask_out,
                  sft_v, idxi_v, idxj_v, ss_v,
                  ri_v, rj_v, vx_v, vy_v, vz_v, sod_v, mask_v, sem):
        wid = lax.axis_index("s") * _NC + lax.axis_index("c")
        iota = lax.iota(jnp.int32, 16)
        c0 = jnp.zeros((16,), jnp.int32)
        c1 = jnp.full((16,), 1, jnp.int32)
        c2 = jnp.full((16,), 2, jnp.int32)

        pltpu.sync_copy(sft_xyz, sft_v)

        rows0 = wid * (n_chunks * _NSUB)

        def chunk_body(c, _):
            row0 = rows0 + c * _NSUB
            base = row0 * _SUB
            pltpu.sync_copy(ii.at[pl.ds(row0, _NSUB)], idxi_v)
            pltpu.sync_copy(jj.at[pl.ds(row0, _NSUB)], idxj_v)
            pltpu.sync_copy(ss.at[pl.ds(base, _CHUNK)], ss_v)

            def gstart(j, carry):
                pltpu.make_async_copy(
                    pos4.at[idxi_v.at[j]], ri_v.at[pl.ds(j * _SUB, _SUB)], sem
                ).start()
                pltpu.make_async_copy(
                    pos4.at[idxj_v.at[j]], rj_v.at[pl.ds(j * _SUB, _SUB)], sem
                ).start()
                return carry

            lax.fori_loop(0, _NSUB, gstart, 0)

            def gwait(j, carry):
                pltpu.make_async_copy(
                    pos4.at[idxi_v.at[j]], ri_v.at[pl.ds(j * _SUB, _SUB)], sem
                ).wait()
                pltpu.make_async_copy(
                    pos4.at[idxj_v.at[j]], rj_v.at[pl.ds(j * _SUB, _SUB)], sem
                ).wait()
                return carry

            lax.fori_loop(0, _NSUB, gwait, 0)

            def grp(t, carry):
                e16 = iota + t * 16
                ssv = ss_v[pl.ds(t * 16, 16)]
                xi = plsc.load_gather(ri_v, [e16, c0])
                yi = plsc.load_gather(ri_v, [e16, c1])
                zi = plsc.load_gather(ri_v, [e16, c2])
                xj = plsc.load_gather(rj_v, [e16, c0])
                yj = plsc.load_gather(rj_v, [e16, c1])
                zj = plsc.load_gather(rj_v, [e16, c2])
                sx = plsc.load_gather(sft_v, [ssv, c0])
                sy = plsc.load_gather(sft_v, [ssv, c1])
                sz = plsc.load_gather(sft_v, [ssv, c2])
                vx = (xj + sx) - xi
                vy = (yj + sy) - yi
                vz = (zj + sz) - zi
                sod = (vx * vx + vy * vy) + vz * vz
                m = sod < _RC2
                zf = jnp.zeros((16,), jnp.float32)
                sod_v[pl.ds(t * 16, 16)] = jnp.where(m, sod, zf)
                mask_v[pl.ds(t * 16, 16)] = m.astype(jnp.int32)
                vx_v[pl.ds(t * 16, 16)] = jnp.where(m, vx, zf)
                vy_v[pl.ds(t * 16, 16)] = jnp.where(m, vy, zf)
                vz_v[pl.ds(t * 16, 16)] = jnp.where(m, vz, zf)
                return carry

            lax.fori_loop(0, _CHUNK // 16, grp, 0)

            pltpu.sync_copy(vx_v, vx_out.at[pl.ds(base, _CHUNK)])
            pltpu.sync_copy(vy_v, vy_out.at[pl.ds(base, _CHUNK)])
            pltpu.sync_copy(vz_v, vz_out.at[pl.ds(base, _CHUNK)])
            pltpu.sync_copy(sod_v, sod_out.at[pl.ds(base, _CHUNK)])
            pltpu.sync_copy(mask_v, mask_out.at[pl.ds(base, _CHUNK)])
            return _

        lax.fori_loop(0, n_chunks, chunk_body, 0)

    return sc_kernel


def kernel(pos_xyz, cel_mat, sft_cel, idx_i, idx_j, idx_s):
    N = pos_xyz.shape[0]
    E = idx_i.shape[0]
    pos4 = jnp.pad(pos_xyz.astype(jnp.float32), ((0, 0), (0, 13)))
    sft_xyz = sft_cel.astype(jnp.float32) @ cel_mat.astype(jnp.float32)
    sft_pad = jnp.pad(sft_xyz, ((0, 32 - sft_xyz.shape[0]), (0, 0)))
    ii = idx_i.astype(jnp.int32).reshape(E // _SUB, _SUB)
    jj = idx_j.astype(jnp.int32).reshape(E // _SUB, _SUB)
    ss = idx_s.astype(jnp.int32)
    sc = _make_sc_kernel(E, N)
    vx, vy, vz, sod, mask_i = sc(pos4, sft_pad, ii, jj, ss)
    vec = jnp.stack((vx, vy, vz), axis=-1)
    return vec, sod, mask_i.astype(jnp.bool_)


# speedup vs baseline: 60.7722x; 1.0802x over previous
import functools

import jax
import jax.numpy as jnp
from jax import lax
from jax.experimental import pallas as pl
from jax.experimental.pallas import tpu as pltpu
from jax.experimental.pallas import tpu_sc as plsc

_NC = 2
_NS = 16
_SUB = 80
_NSUB = 25
_CHUNK = _SUB * _NSUB
_RC2 = 36.0


def _make_sc_kernel(E, N):
    NW = _NC * _NS
    per_worker = E // NW
    n_chunks = per_worker // _CHUNK
    n_rows = E // _SUB

    mesh = plsc.VectorSubcoreMesh(
        core_axis_name="c", subcore_axis_name="s", num_cores=_NC
    )

    @functools.partial(
        pl.kernel,
        out_type=[
            jax.ShapeDtypeStruct((E,), jnp.float32),
            jax.ShapeDtypeStruct((E,), jnp.float32),
            jax.ShapeDtypeStruct((E,), jnp.float32),
            jax.ShapeDtypeStruct((E,), jnp.float32),
            jax.ShapeDtypeStruct((E,), jnp.int32),
        ],
        mesh=mesh,
        compiler_params=pltpu.CompilerParams(
            needs_layout_passes=False, use_tc_tiling_on_sc=False
        ),
        scratch_types=[
            pltpu.VMEM((32, 3), jnp.float32),
            pltpu.VMEM((_NSUB, _SUB), jnp.int32),
            pltpu.VMEM((_NSUB, _SUB), jnp.int32),
            pltpu.VMEM((_CHUNK,), jnp.int32),
            pltpu.VMEM((_CHUNK, 8), jnp.float32),
            pltpu.VMEM((_CHUNK, 8), jnp.float32),
            pltpu.VMEM((_CHUNK,), jnp.float32),
            pltpu.VMEM((_CHUNK,), jnp.float32),
            pltpu.VMEM((_CHUNK,), jnp.float32),
            pltpu.VMEM((_CHUNK,), jnp.float32),
            pltpu.VMEM((_CHUNK,), jnp.int32),
            pltpu.SemaphoreType.DMA,
        ],
    )
    def sc_kernel(pos4, sft_xyz, ii, jj, ss,
                  vx_out, vy_out, vz_out, sod_out, mask_out,
                  sft_v, idxi_v, idxj_v, ss_v,
                  ri_v, rj_v, vx_v, vy_v, vz_v, sod_v, mask_v, sem):
        wid = lax.axis_index("s") * _NC + lax.axis_index("c")
        iota = lax.iota(jnp.int32, 16)
        c0 = jnp.zeros((16,), jnp.int32)
        c1 = jnp.full((16,), 1, jnp.int32)
        c2 = jnp.full((16,), 2, jnp.int32)

        pltpu.sync_copy(sft_xyz, sft_v)

        rows0 = wid * (n_chunks * _NSUB)

        def chunk_body(c, _):
            row0 = rows0 + c * _NSUB
            base = row0 * _SUB
            pltpu.sync_copy(ii.at[pl.ds(row0, _NSUB)], idxi_v)
            pltpu.sync_copy(jj.at[pl.ds(row0, _NSUB)], idxj_v)
            pltpu.sync_copy(ss.at[pl.ds(base, _CHUNK)], ss_v)

            def gstart(j, carry):
                pltpu.make_async_copy(
                    pos4.at[idxi_v.at[j]], ri_v.at[pl.ds(j * _SUB, _SUB)], sem
                ).start()
                pltpu.make_async_copy(
                    pos4.at[idxj_v.at[j]], rj_v.at[pl.ds(j * _SUB, _SUB)], sem
                ).start()
                return carry

            lax.fori_loop(0, _NSUB, gstart, 0)

            def gwait(j, carry):
                pltpu.make_async_copy(
                    pos4.at[idxi_v.at[j]], ri_v.at[pl.ds(j * _SUB, _SUB)], sem
                ).wait()
                pltpu.make_async_copy(
                    pos4.at[idxj_v.at[j]], rj_v.at[pl.ds(j * _SUB, _SUB)], sem
                ).wait()
                return carry

            lax.fori_loop(0, _NSUB, gwait, 0)

            def grp(t, carry):
                e16 = iota + t * 16
                ssv = ss_v[pl.ds(t * 16, 16)]
                xi = plsc.load_gather(ri_v, [e16, c0])
                yi = plsc.load_gather(ri_v, [e16, c1])
                zi = plsc.load_gather(ri_v, [e16, c2])
                xj = plsc.load_gather(rj_v, [e16, c0])
                yj = plsc.load_gather(rj_v, [e16, c1])
                zj = plsc.load_gather(rj_v, [e16, c2])
                sx = plsc.load_gather(sft_v, [ssv, c0])
                sy = plsc.load_gather(sft_v, [ssv, c1])
                sz = plsc.load_gather(sft_v, [ssv, c2])
                vx = (xj + sx) - xi
                vy = (yj + sy) - yi
                vz = (zj + sz) - zi
                sod = (vx * vx + vy * vy) + vz * vz
                m = sod < _RC2
                zf = jnp.zeros((16,), jnp.float32)
                sod_v[pl.ds(t * 16, 16)] = jnp.where(m, sod, zf)
                mask_v[pl.ds(t * 16, 16)] = m.astype(jnp.int32)
                vx_v[pl.ds(t * 16, 16)] = jnp.where(m, vx, zf)
                vy_v[pl.ds(t * 16, 16)] = jnp.where(m, vy, zf)
                vz_v[pl.ds(t * 16, 16)] = jnp.where(m, vz, zf)
                return carry

            lax.fori_loop(0, _CHUNK // 16, grp, 0)

            pltpu.sync_copy(vx_v, vx_out.at[pl.ds(base, _CHUNK)])
            pltpu.sync_copy(vy_v, vy_out.at[pl.ds(base, _CHUNK)])
            pltpu.sync_copy(vz_v, vz_out.at[pl.ds(base, _CHUNK)])
            pltpu.sync_copy(sod_v, sod_out.at[pl.ds(base, _CHUNK)])
            pltpu.sync_copy(mask_v, mask_out.at[pl.ds(base, _CHUNK)])
            return _

        lax.fori_loop(0, n_chunks, chunk_body, 0)

    return sc_kernel


def kernel(pos_xyz, cel_mat, sft_cel, idx_i, idx_j, idx_s):
    N = pos_xyz.shape[0]
    E = idx_i.shape[0]
    pos4 = jnp.pad(pos_xyz.astype(jnp.float32), ((0, 0), (0, 5)))
    sft_xyz = sft_cel.astype(jnp.float32) @ cel_mat.astype(jnp.float32)
    sft_pad = jnp.pad(sft_xyz, ((0, 32 - sft_xyz.shape[0]), (0, 0)))
    ii = idx_i.astype(jnp.int32).reshape(E // _SUB, _SUB)
    jj = idx_j.astype(jnp.int32).reshape(E // _SUB, _SUB)
    ss = idx_s.astype(jnp.int32)
    sc = _make_sc_kernel(E, N)
    vx, vy, vz, sod, mask_i = sc(pos4, sft_pad, ii, jj, ss)
    vec = jnp.stack((vx, vy, vz), axis=-1)
    return vec, sod, mask_i.astype(jnp.bool_)


# speedup vs baseline: 83.4837x; 1.3737x over previous
import functools

import jax
import jax.numpy as jnp
from jax import lax
from jax.experimental import pallas as pl
from jax.experimental.pallas import tpu as pltpu
from jax.experimental.pallas import tpu_sc as plsc

_NC = 2
_NS = 16
_SUB = 80
_NSUB = 25
_CHUNK = _SUB * _NSUB
_RC2 = 36.0


def _make_sc_kernel(E, N):
    NW = _NC * _NS
    per_worker = E // NW
    n_chunks = per_worker // _CHUNK
    assert n_chunks % 2 == 0 and n_chunks >= 2

    mesh = plsc.VectorSubcoreMesh(
        core_axis_name="c", subcore_axis_name="s", num_cores=_NC
    )

    chunk_bufs = [
        pltpu.VMEM((_NSUB, _SUB), jnp.int32),
        pltpu.VMEM((_NSUB, _SUB), jnp.int32),
        pltpu.VMEM((_CHUNK,), jnp.int32),
        pltpu.VMEM((_CHUNK, 8), jnp.float32),
        pltpu.VMEM((_CHUNK, 8), jnp.float32),
        pltpu.SemaphoreType.DMA,
    ]

    @functools.partial(
        pl.kernel,
        out_type=[
            jax.ShapeDtypeStruct((E,), jnp.float32),
            jax.ShapeDtypeStruct((E,), jnp.float32),
            jax.ShapeDtypeStruct((E,), jnp.float32),
            jax.ShapeDtypeStruct((E,), jnp.float32),
            jax.ShapeDtypeStruct((E,), jnp.int32),
        ],
        mesh=mesh,
        compiler_params=pltpu.CompilerParams(
            needs_layout_passes=False, use_tc_tiling_on_sc=False
        ),
        scratch_types=[
            pltpu.VMEM((32, 3), jnp.float32),
            *chunk_bufs,
            *chunk_bufs,
            pltpu.VMEM((_CHUNK,), jnp.float32),
            pltpu.VMEM((_CHUNK,), jnp.float32),
            pltpu.VMEM((_CHUNK,), jnp.float32),
            pltpu.VMEM((_CHUNK,), jnp.float32),
            pltpu.VMEM((_CHUNK,), jnp.int32),
        ],
    )
    def sc_kernel(pos8, sft_xyz, ii, jj, ss,
                  vx_out, vy_out, vz_out, sod_out, mask_out,
                  sft_v,
                  idxi_a, idxj_a, ss_a, ri_a, rj_a, sem_a,
                  idxi_b, idxj_b, ss_b, ri_b, rj_b, sem_b,
                  vx_v, vy_v, vz_v, sod_v, mask_v):
        wid = lax.axis_index("s") * _NC + lax.axis_index("c")
        iota = lax.iota(jnp.int32, 16)
        c0 = jnp.zeros((16,), jnp.int32)
        c1 = jnp.full((16,), 1, jnp.int32)
        c2 = jnp.full((16,), 2, jnp.int32)

        pltpu.sync_copy(sft_xyz, sft_v)

        rows0 = wid * (n_chunks * _NSUB)

        slot_a = (idxi_a, idxj_a, ss_a, ri_a, rj_a, sem_a)
        slot_b = (idxi_b, idxj_b, ss_b, ri_b, rj_b, sem_b)

        def fire(c, slot):
            idxi_v, idxj_v, ss_v, ri_v, rj_v, sem = slot
            row0 = rows0 + c * _NSUB
            base = row0 * _SUB
            pltpu.sync_copy(ii.at[pl.ds(row0, _NSUB)], idxi_v)
            pltpu.sync_copy(jj.at[pl.ds(row0, _NSUB)], idxj_v)
            pltpu.sync_copy(ss.at[pl.ds(base, _CHUNK)], ss_v)

            def gstart(j, carry):
                pltpu.make_async_copy(
                    pos8.at[idxi_v.at[j]], ri_v.at[pl.ds(j * _SUB, _SUB)], sem
                ).start()
                pltpu.make_async_copy(
                    pos8.at[idxj_v.at[j]], rj_v.at[pl.ds(j * _SUB, _SUB)], sem
                ).start()
                return carry

            lax.fori_loop(0, _NSUB, gstart, 0)

        def process(c, slot):
            idxi_v, idxj_v, ss_v, ri_v, rj_v, sem = slot
            base = (rows0 + c * _NSUB) * _SUB

            def gwait(j, carry):
                pltpu.make_async_copy(
                    pos8.at[idxi_v.at[j]], ri_v.at[pl.ds(j * _SUB, _SUB)], sem
                ).wait()
                pltpu.make_async_copy(
                    pos8.at[idxj_v.at[j]], rj_v.at[pl.ds(j * _SUB, _SUB)], sem
                ).wait()
                return carry

            lax.fori_loop(0, _NSUB, gwait, 0)

            def grp(t, carry):
                e16 = iota + t * 16
                ssv = ss_v[pl.ds(t * 16, 16)]
                xi = plsc.load_gather(ri_v, [e16, c0])
                yi = plsc.load_gather(ri_v, [e16, c1])
                zi = plsc.load_gather(ri_v, [e16, c2])
                xj = plsc.load_gather(rj_v, [e16, c0])
                yj = plsc.load_gather(rj_v, [e16, c1])
                zj = plsc.load_gather(rj_v, [e16, c2])
                sx = plsc.load_gather(sft_v, [ssv, c0])
                sy = plsc.load_gather(sft_v, [ssv, c1])
                sz = plsc.load_gather(sft_v, [ssv, c2])
                vx = (xj + sx) - xi
                vy = (yj + sy) - yi
                vz = (zj + sz) - zi
                sod = (vx * vx + vy * vy) + vz * vz
                m = sod < _RC2
                zf = jnp.zeros((16,), jnp.float32)
                sod_v[pl.ds(t * 16, 16)] = jnp.where(m, sod, zf)
                mask_v[pl.ds(t * 16, 16)] = m.astype(jnp.int32)
                vx_v[pl.ds(t * 16, 16)] = jnp.where(m, vx, zf)
                vy_v[pl.ds(t * 16, 16)] = jnp.where(m, vy, zf)
                vz_v[pl.ds(t * 16, 16)] = jnp.where(m, vz, zf)
                return carry

            lax.fori_loop(0, _CHUNK // 16, grp, 0)

            pltpu.sync_copy(vx_v, vx_out.at[pl.ds(base, _CHUNK)])
            pltpu.sync_copy(vy_v, vy_out.at[pl.ds(base, _CHUNK)])
            pltpu.sync_copy(vz_v, vz_out.at[pl.ds(base, _CHUNK)])
            pltpu.sync_copy(sod_v, sod_out.at[pl.ds(base, _CHUNK)])
            pltpu.sync_copy(mask_v, mask_out.at[pl.ds(base, _CHUNK)])

        fire(jnp.int32(0), slot_a)

        def pair_body(k, carry):
            ca = 2 * k
            fire(ca + 1, slot_b)
            process(ca, slot_a)
            fire(ca + 2, slot_a)
            process(ca + 1, slot_b)
            return carry

        lax.fori_loop(0, n_chunks // 2 - 1, pair_body, 0)

        last = jnp.int32(n_chunks - 2)
        fire(last + 1, slot_b)
        process(last, slot_a)
        process(last + 1, slot_b)

    return sc_kernel


def kernel(pos_xyz, cel_mat, sft_cel, idx_i, idx_j, idx_s):
    N = pos_xyz.shape[0]
    E = idx_i.shape[0]
    pos8 = jnp.pad(pos_xyz.astype(jnp.float32), ((0, 0), (0, 5)))
    sft_xyz = sft_cel.astype(jnp.float32) @ cel_mat.astype(jnp.float32)
    sft_pad = jnp.pad(sft_xyz, ((0, 32 - sft_xyz.shape[0]), (0, 0)))
    ii = idx_i.astype(jnp.int32).reshape(E // _SUB, _SUB)
    jj = idx_j.astype(jnp.int32).reshape(E // _SUB, _SUB)
    ss = idx_s.astype(jnp.int32)
    sc = _make_sc_kernel(E, N)
    vx, vy, vz, sod, mask_i = sc(pos8, sft_pad, ii, jj, ss)
    vec = jnp.stack((vx, vy, vz), axis=-1)
    return vec, sod, mask_i.astype(jnp.bool_)


# speedup vs baseline: 87.1627x; 1.0441x over previous
import functools

import jax
import jax.numpy as jnp
from jax import lax
from jax.experimental import pallas as pl
from jax.experimental.pallas import tpu as pltpu
from jax.experimental.pallas import tpu_sc as plsc

_NC = 2
_NS = 16
_SUB = 80
_NSUB = 25
_CHUNK = _SUB * _NSUB
_RC2 = 36.0


def _make_sc_kernel(E, N):
    NW = _NC * _NS
    per_worker = E // NW
    n_chunks = per_worker // _CHUNK
    assert n_chunks % 2 == 0 and n_chunks >= 4

    mesh = plsc.VectorSubcoreMesh(
        core_axis_name="c", subcore_axis_name="s", num_cores=_NC
    )

    slot_bufs = [
        pltpu.VMEM((_NSUB, _SUB), jnp.int32),
        pltpu.VMEM((_NSUB, _SUB), jnp.int32),
        pltpu.VMEM((_CHUNK,), jnp.int32),
        pltpu.VMEM((_CHUNK, 8), jnp.float32),
        pltpu.VMEM((_CHUNK, 8), jnp.float32),
        pltpu.SemaphoreType.DMA,
        pltpu.VMEM((_CHUNK,), jnp.float32),
        pltpu.VMEM((_CHUNK,), jnp.float32),
        pltpu.VMEM((_CHUNK,), jnp.float32),
        pltpu.VMEM((_CHUNK,), jnp.float32),
        pltpu.VMEM((_CHUNK,), jnp.int32),
        pltpu.SemaphoreType.DMA,
    ]

    @functools.partial(
        pl.kernel,
        out_type=[
            jax.ShapeDtypeStruct((E,), jnp.float32),
            jax.ShapeDtypeStruct((E,), jnp.float32),
            jax.ShapeDtypeStruct((E,), jnp.float32),
            jax.ShapeDtypeStruct((E,), jnp.float32),
            jax.ShapeDtypeStruct((E,), jnp.int32),
        ],
        mesh=mesh,
        compiler_params=pltpu.CompilerParams(
            needs_layout_passes=False, use_tc_tiling_on_sc=False
        ),
        scratch_types=[
            pltpu.VMEM((32, 3), jnp.float32),
            *slot_bufs,
            *slot_bufs,
        ],
    )
    def sc_kernel(pos8, sft_xyz, ii, jj, ss,
                  vx_out, vy_out, vz_out, sod_out, mask_out,
                  sft_v, *slots_flat):
        wid = lax.axis_index("s") * _NC + lax.axis_index("c")
        iota = lax.iota(jnp.int32, 16)
        c0 = jnp.zeros((16,), jnp.int32)
        c1 = jnp.full((16,), 1, jnp.int32)
        c2 = jnp.full((16,), 2, jnp.int32)

        ns = len(slot_bufs)
        slot_a = slots_flat[:ns]
        slot_b = slots_flat[ns:]

        pltpu.sync_copy(sft_xyz, sft_v)

        rows0 = wid * (n_chunks * _NSUB)

        def out_copies(c, slot):
            (_, _, _, _, _, _,
             vx_v, vy_v, vz_v, sod_v, mask_v, sem_w) = slot
            base = (rows0 + c * _NSUB) * _SUB
            return [
                pltpu.make_async_copy(
                    vx_v, vx_out.at[pl.ds(base, _CHUNK)], sem_w),
                pltpu.make_async_copy(
                    vy_v, vy_out.at[pl.ds(base, _CHUNK)], sem_w),
                pltpu.make_async_copy(
                    vz_v, vz_out.at[pl.ds(base, _CHUNK)], sem_w),
                pltpu.make_async_copy(
                    sod_v, sod_out.at[pl.ds(base, _CHUNK)], sem_w),
                pltpu.make_async_copy(
                    mask_v, mask_out.at[pl.ds(base, _CHUNK)], sem_w),
            ]

        def fire(c, slot):
            idxi_v, idxj_v, ss_v, ri_v, rj_v, sem = slot[:6]
            row0 = rows0 + c * _NSUB
            base = row0 * _SUB
            pltpu.sync_copy(ii.at[pl.ds(row0, _NSUB)], idxi_v)
            pltpu.sync_copy(jj.at[pl.ds(row0, _NSUB)], idxj_v)
            pltpu.sync_copy(ss.at[pl.ds(base, _CHUNK)], ss_v)

            def gstart(j, carry):
                pltpu.make_async_copy(
                    pos8.at[idxi_v.at[j]], ri_v.at[pl.ds(j * _SUB, _SUB)], sem
                ).start()
                pltpu.make_async_copy(
                    pos8.at[idxj_v.at[j]], rj_v.at[pl.ds(j * _SUB, _SUB)], sem
                ).start()
                return carry

            lax.fori_loop(0, _NSUB, gstart, 0)

        def process(c, slot, first):
            (idxi_v, idxj_v, ss_v, ri_v, rj_v, sem,
             vx_v, vy_v, vz_v, sod_v, mask_v, sem_w) = slot

            def gwait(j, carry):
                pltpu.make_async_copy(
                    pos8.at[idxi_v.at[j]], ri_v.at[pl.ds(j * _SUB, _SUB)], sem
                ).wait()
                pltpu.make_async_copy(
                    pos8.at[idxj_v.at[j]], rj_v.at[pl.ds(j * _SUB, _SUB)], sem
                ).wait()
                return carry

            lax.fori_loop(0, _NSUB, gwait, 0)

            if not first:
                for cp in out_copies(c - 2, slot):
                    cp.wait()

            def grp(t, carry):
                e16 = iota + t * 16
                ssv = ss_v[pl.ds(t * 16, 16)]
                xi = plsc.load_gather(ri_v, [e16, c0])
                yi = plsc.load_gather(ri_v, [e16, c1])
                zi = plsc.load_gather(ri_v, [e16, c2])
                xj = plsc.load_gather(rj_v, [e16, c0])
                yj = plsc.load_gather(rj_v, [e16, c1])
                zj = plsc.load_gather(rj_v, [e16, c2])
                sx = plsc.load_gather(sft_v, [ssv, c0])
                sy = plsc.load_gather(sft_v, [ssv, c1])
                sz = plsc.load_gather(sft_v, [ssv, c2])
                vx = (xj + sx) - xi
                vy = (yj + sy) - yi
                vz = (zj + sz) - zi
                sod = (vx * vx + vy * vy) + vz * vz
                m = sod < _RC2
                zf = jnp.zeros((16,), jnp.float32)
                sod_v[pl.ds(t * 16, 16)] = jnp.where(m, sod, zf)
                mask_v[pl.ds(t * 16, 16)] = m.astype(jnp.int32)
                vx_v[pl.ds(t * 16, 16)] = jnp.where(m, vx, zf)
                vy_v[pl.ds(t * 16, 16)] = jnp.where(m, vy, zf)
                vz_v[pl.ds(t * 16, 16)] = jnp.where(m, vz, zf)
                return carry

            lax.fori_loop(0, _CHUNK // 16, grp, 0)

            for cp in out_copies(c, slot):
                cp.start()

        fire(jnp.int32(0), slot_a)
        fire(jnp.int32(1), slot_b)
        process(jnp.int32(0), slot_a, first=True)
        fire(jnp.int32(2), slot_a)
        process(jnp.int32(1), slot_b, first=True)
        fire(jnp.int32(3), slot_b)

        def pair_body(k, carry):
            ca = 2 * k
            process(ca, slot_a, first=False)
            fire(ca + 2, slot_a)
            process(ca + 1, slot_b, first=False)
            fire(ca + 3, slot_b)
            return carry

        lax.fori_loop(1, n_chunks // 2 - 1, pair_body, 0)

        last = jnp.int32(n_chunks - 2)
        process(last, slot_a, first=False)
        process(last + 1, slot_b, first=False)
        for cp in out_copies(last, slot_a):
            cp.wait()
        for cp in out_copies(last + 1, slot_b):
            cp.wait()

    return sc_kernel


def kernel(pos_xyz, cel_mat, sft_cel, idx_i, idx_j, idx_s):
    N = pos_xyz.shape[0]
    E = idx_i.shape[0]
    pos8 = jnp.pad(pos_xyz.astype(jnp.float32), ((0, 0), (0, 5)))
    sft_xyz = sft_cel.astype(jnp.float32) @ cel_mat.astype(jnp.float32)
    sft_pad = jnp.pad(sft_xyz, ((0, 32 - sft_xyz.shape[0]), (0, 0)))
    ii = idx_i.astype(jnp.int32).reshape(E // _SUB, _SUB)
    jj = idx_j.astype(jnp.int32).reshape(E // _SUB, _SUB)
    ss = idx_s.astype(jnp.int32)
    sc = _make_sc_kernel(E, N)
    vx, vy, vz, sod, mask_i = sc(pos8, sft_pad, ii, jj, ss)
    vec = jnp.stack((vx, vy, vz), axis=-1)
    return vec, sod, mask_i.astype(jnp.bool_)


# speedup vs baseline: 87.3863x; 1.0026x over previous
import functools

import jax
import jax.numpy as jnp
from jax import lax
from jax.experimental import pallas as pl
from jax.experimental.pallas import tpu as pltpu
from jax.experimental.pallas import tpu_sc as plsc

_NC = 2
_NS = 16
_SUB = 80
_NSUB = 25
_CHUNK = _SUB * _NSUB
_RC2 = 36.0


def _make_sc_kernel(E, N):
    NW = _NC * _NS
    per_worker = E // NW
    n_chunks = per_worker // _CHUNK
    assert n_chunks % 2 == 0 and n_chunks >= 4

    mesh = plsc.VectorSubcoreMesh(
        core_axis_name="c", subcore_axis_name="s", num_cores=_NC
    )

    slot_bufs = [
        pltpu.VMEM((_NSUB, _SUB), jnp.int32),
        pltpu.VMEM((_NSUB, _SUB), jnp.int32),
        pltpu.VMEM((_CHUNK,), jnp.int32),
        pltpu.VMEM((_CHUNK, 8), jnp.float32),
        pltpu.VMEM((_CHUNK, 8), jnp.float32),
        pltpu.SemaphoreType.DMA,
        pltpu.VMEM((_CHUNK,), jnp.float32),
        pltpu.VMEM((_CHUNK,), jnp.float32),
        pltpu.VMEM((_CHUNK,), jnp.float32),
        pltpu.VMEM((_CHUNK,), jnp.float32),
        pltpu.VMEM((_CHUNK,), jnp.int32),
        pltpu.SemaphoreType.DMA,
    ]

    @functools.partial(
        pl.kernel,
        out_type=[
            jax.ShapeDtypeStruct((E,), jnp.float32),
            jax.ShapeDtypeStruct((E,), jnp.float32),
            jax.ShapeDtypeStruct((E,), jnp.float32),
            jax.ShapeDtypeStruct((E,), jnp.float32),
            jax.ShapeDtypeStruct((E,), jnp.int32),
        ],
        mesh=mesh,
        compiler_params=pltpu.CompilerParams(
            needs_layout_passes=False, use_tc_tiling_on_sc=False
        ),
        scratch_types=[
            pltpu.VMEM((32, 3), jnp.float32),
            *slot_bufs,
            *slot_bufs,
        ],
    )
    def sc_kernel(pos8, sft_xyz, ii, jj, ss,
                  vx_out, vy_out, vz_out, sod_out, mask_out,
                  sft_v, *slots_flat):
        wid = lax.axis_index("s") * _NC + lax.axis_index("c")
        iota = lax.iota(jnp.int32, 16)
        c0 = jnp.zeros((16,), jnp.int32)
        c1 = jnp.full((16,), 1, jnp.int32)
        c2 = jnp.full((16,), 2, jnp.int32)

        ns = len(slot_bufs)
        slot_a = slots_flat[:ns]
        slot_b = slots_flat[ns:]

        pltpu.sync_copy(sft_xyz, sft_v)

        rows0 = wid * (n_chunks * _NSUB)

        def out_copies(c, slot):
            (_, _, _, _, _, _,
             vx_v, vy_v, vz_v, sod_v, mask_v, sem_w) = slot
            base = (rows0 + c * _NSUB) * _SUB
            return [
                pltpu.make_async_copy(
                    vx_v, vx_out.at[pl.ds(base, _CHUNK)], sem_w),
                pltpu.make_async_copy(
                    vy_v, vy_out.at[pl.ds(base, _CHUNK)], sem_w),
                pltpu.make_async_copy(
                    vz_v, vz_out.at[pl.ds(base, _CHUNK)], sem_w),
                pltpu.make_async_copy(
                    sod_v, sod_out.at[pl.ds(base, _CHUNK)], sem_w),
                pltpu.make_async_copy(
                    mask_v, mask_out.at[pl.ds(base, _CHUNK)], sem_w),
            ]

        def fire(c, slot):
            idxi_v, idxj_v, ss_v, ri_v, rj_v, sem = slot[:6]
            row0 = rows0 + c * _NSUB
            base = row0 * _SUB
            pltpu.sync_copy(ii.at[pl.ds(row0, _NSUB)], idxi_v)
            pltpu.sync_copy(jj.at[pl.ds(row0, _NSUB)], idxj_v)
            pltpu.sync_copy(ss.at[pl.ds(base, _CHUNK)], ss_v)

            def gstart(j, carry):
                pltpu.make_async_copy(
                    pos8.at[idxi_v.at[j]], ri_v.at[pl.ds(j * _SUB, _SUB)], sem
                ).start()
                pltpu.make_async_copy(
                    pos8.at[idxj_v.at[j]], rj_v.at[pl.ds(j * _SUB, _SUB)], sem
                ).start()
                return carry

            lax.fori_loop(0, _NSUB, gstart, 0)

        def process(c, slot, first):
            (idxi_v, idxj_v, ss_v, ri_v, rj_v, sem,
             vx_v, vy_v, vz_v, sod_v, mask_v, sem_w) = slot

            def gwait(j, carry):
                pltpu.make_async_copy(
                    pos8.at[idxi_v.at[j]], ri_v.at[pl.ds(j * _SUB, _SUB)], sem
                ).wait()
                pltpu.make_async_copy(
                    pos8.at[idxj_v.at[j]], rj_v.at[pl.ds(j * _SUB, _SUB)], sem
                ).wait()
                return carry

            lax.fori_loop(0, _NSUB, gwait, 0)

            if not first:
                for cp in out_copies(c - 2, slot):
                    cp.wait()

            def grp(t, carry):
                e16 = iota + t * 16
                ssv = ss_v[pl.ds(t * 16, 16)]
                xi = plsc.load_gather(ri_v, [e16, c0])
                yi = plsc.load_gather(ri_v, [e16, c1])
                zi = plsc.load_gather(ri_v, [e16, c2])
                xj = plsc.load_gather(rj_v, [e16, c0])
                yj = plsc.load_gather(rj_v, [e16, c1])
                zj = plsc.load_gather(rj_v, [e16, c2])
                sx = plsc.load_gather(sft_v, [ssv, c0])
                sy = plsc.load_gather(sft_v, [ssv, c1])
                sz = plsc.load_gather(sft_v, [ssv, c2])
                vx = (xj + sx) - xi
                vy = (yj + sy) - yi
                vz = (zj + sz) - zi
                sod = (vx * vx + vy * vy) + vz * vz
                m = sod < _RC2
                zf = jnp.zeros((16,), jnp.float32)
                sod_v[pl.ds(t * 16, 16)] = jnp.where(m, sod, zf)
                mask_v[pl.ds(t * 16, 16)] = m.astype(jnp.int32)
                vx_v[pl.ds(t * 16, 16)] = jnp.where(m, vx, zf)
                vy_v[pl.ds(t * 16, 16)] = jnp.where(m, vy, zf)
                vz_v[pl.ds(t * 16, 16)] = jnp.where(m, vz, zf)
                return carry

            lax.fori_loop(0, _CHUNK // 16, grp, 0, unroll=4)

            for cp in out_copies(c, slot):
                cp.start()

        fire(jnp.int32(0), slot_a)
        fire(jnp.int32(1), slot_b)
        process(jnp.int32(0), slot_a, first=True)
        fire(jnp.int32(2), slot_a)
        process(jnp.int32(1), slot_b, first=True)
        fire(jnp.int32(3), slot_b)

        def pair_body(k, carry):
            ca = 2 * k
            process(ca, slot_a, first=False)
            fire(ca + 2, slot_a)
            process(ca + 1, slot_b, first=False)
            fire(ca + 3, slot_b)
            return carry

        lax.fori_loop(1, n_chunks // 2 - 1, pair_body, 0)

        last = jnp.int32(n_chunks - 2)
        process(last, slot_a, first=False)
        process(last + 1, slot_b, first=False)
        for cp in out_copies(last, slot_a):
            cp.wait()
        for cp in out_copies(last + 1, slot_b):
            cp.wait()

    return sc_kernel


def kernel(pos_xyz, cel_mat, sft_cel, idx_i, idx_j, idx_s):
    N = pos_xyz.shape[0]
    E = idx_i.shape[0]
    pos8 = jnp.pad(pos_xyz.astype(jnp.float32), ((0, 0), (0, 5)))
    sft_xyz = sft_cel.astype(jnp.float32) @ cel_mat.astype(jnp.float32)
    sft_pad = jnp.pad(sft_xyz, ((0, 32 - sft_xyz.shape[0]), (0, 0)))
    ii = idx_i.astype(jnp.int32).reshape(E // _SUB, _SUB)
    jj = idx_j.astype(jnp.int32).reshape(E // _SUB, _SUB)
    ss = idx_s.astype(jnp.int32)
    sc = _make_sc_kernel(E, N)
    vx, vy, vz, sod, mask_i = sc(pos8, sft_pad, ii, jj, ss)
    vec = jnp.stack((vx, vy, vz), axis=-1)
    return vec, sod, mask_i.astype(jnp.bool_)
